# baseline (speedup 1.0000x reference)
import numpy as np
import jax
import jax.numpy as jnp
from jax.experimental import pallas as pl

_HW = 196
_SEG = 6272
_CPS = _SEG // _HW
_ROWS_PER_SAMPLE = None
_BB = 8


def _seg_sum_kernel(x_ref, e_ref, out_ref):
    xb = x_ref[...]
    hi_bf = xb.astype(jnp.bfloat16)
    lo = xb - hi_bf.astype(jnp.float32)
    eb = e_ref[...]
    acc = (jnp.dot(hi_bf, eb, preferred_element_type=jnp.float32)
           + jnp.dot(lo.astype(jnp.bfloat16), eb,
                     preferred_element_type=jnp.float32))
    out_ref[...] = acc * (1.0 / _HW)


def _finish_kernel(sem_ref, cr_ref, snr_ref, w1_ref, b1_ref, w2t_ref,
                   b2_ref, ch0_ref, wctc_ref, wcts_ref, wst_ref, wot_ref,
                   wclst_ref, bcls_ref, out_ref):
    sem = sem_ref[...]
    C = sem.shape[1]

    h1 = jnp.maximum(snr_ref[0, 0] * w1_ref[...] + b1_ref[...], 0.0)
    sv = jnp.maximum(
        jnp.dot(h1, w2t_ref[...], preferred_element_type=jnp.float32)
        + b2_ref[...], 0.0)
    contrib = (
        jnp.dot(ch0_ref[...], wctc_ref[...], preferred_element_type=jnp.float32)
        + jnp.dot(sv, wcts_ref[...], preferred_element_type=jnp.float32))

    hid = jnp.maximum(
        jnp.dot(sem, wst_ref[...], preferred_element_type=jnp.float32)
        + contrib, 0.0)
    wts = jax.nn.sigmoid(
        jnp.dot(hid, wot_ref[...], preferred_element_type=jnp.float32))

    cr_c = jnp.clip(cr_ref[...], 0.001, 1.0)
    k = jnp.clip(jnp.round(cr_c * C), 1.0, float(C)).astype(jnp.int32)

    bits = jax.lax.bitcast_convert_type(wts, jnp.int32)

    def body(i, t):
        cand = t | (jnp.int32(1) << (jnp.int32(30) - i))
        cnt = jnp.sum((bits >= cand).astype(jnp.int32), axis=1, keepdims=True)
        return jnp.where(cnt >= k, cand, t)

    t = jax.lax.fori_loop(0, 31, body, jnp.zeros_like(k))
    mask = (bits >= t).astype(jnp.float32)

    pooled = sem * wts * mask
    out_ref[...] = (
        jnp.dot(pooled, wclst_ref[...], preferred_element_type=jnp.float32)
        + bcls_ref[...])


def kernel(x, snr_db, cr, channel_embed, snr_w1, snr_b1, snr_w2, snr_b2,
           Ws, Wc, Wo, Wcls, bcls):
    B, C, H, W = x.shape
    hw = H * W
    assert hw == _HW and (C * hw) % _SEG == 0
    rows = (C * hw) // _SEG
    x2 = x.reshape(B * rows, _SEG)

    u = np.arange(_SEG)
    e_np = np.zeros((_SEG, _CPS), dtype=np.float32)
    e_np[u, u // _HW] = 1.0
    e_mat = jnp.asarray(e_np, dtype=jnp.bfloat16)

    sem24 = pl.pallas_call(
        _seg_sum_kernel,
        grid=(B // _BB,),
        in_specs=[
            pl.BlockSpec((_BB * rows, _SEG), lambda i: (i, 0)),
            pl.BlockSpec((_SEG, _CPS), lambda i: (0, 0)),
        ],
        out_specs=pl.BlockSpec((_BB * rows, _CPS), lambda i: (i, 0)),
        out_shape=jax.ShapeDtypeStruct((B * rows, _CPS), x.dtype),
    )(x2, e_mat)
    sem = sem24.reshape(B, C)

    crr = cr.reshape(B, 1)
    snr_sc = (jnp.asarray(snr_db, dtype=x.dtype) / 28.0).reshape(1, 1)
    w1r = snr_w1.T
    b1r = snr_b1.reshape(1, -1)
    w2t = snr_w2.T
    b2r = snr_b2.reshape(1, -1)
    ch0 = channel_embed[0].reshape(1, -1)
    e = channel_embed.shape[1]
    wct = Wc.T
    wctc, wcts = wct[:e], wct[e:]
    wst = Ws.T
    wot = Wo.T
    wclst = Wcls.T
    bclsr = bcls.reshape(1, -1)
    n_cls = Wcls.shape[0]
    hidden = Ws.shape[0]

    full = lambda shape: pl.BlockSpec(shape, lambda: (0,) * len(shape))
    return pl.pallas_call(
        _finish_kernel,
        in_specs=[
            full((B, C)),
            full((B, 1)),
            full((1, 1)),
            full((1, e)),
            full((1, e)),
            full((e, e)),
            full((1, e)),
            full((1, e)),
            full((e, hidden)),
            full((e, hidden)),
            full((C, hidden)),
            full((hidden, C)),
            full((C, n_cls)),
            full((1, n_cls)),
        ],
        out_specs=full((B, n_cls)),
        out_shape=jax.ShapeDtypeStruct((B, n_cls), x.dtype),
    )(sem, crr, snr_sc, w1r, b1r, w2t, b2r, ch0, wctc, wcts, wst, wot,
      wclst, bclsr)


# speedup vs baseline: 13.7375x; 13.7375x over previous
import jax
import jax.numpy as jnp
from jax.experimental import pallas as pl

_HW = 196
_HB = 2


def _sem_kernel(xt_ref, out_ref):
    i = pl.program_id(0)

    @pl.when(i == 0)
    def _():
        out_ref[...] = jnp.zeros_like(out_ref)

    acc = out_ref[...]
    xb = xt_ref[...]
    for w in range(14):
        for h in range(_HB):
            acc = acc + xb[h, w]
    out_ref[...] = acc


def _finish_kernel(sem_ref, cr_ref, snr_ref, w1_ref, b1_ref, w2t_ref,
                   b2_ref, ch0_ref, wctc_ref, wcts_ref, wst_ref, wot_ref,
                   wclst_ref, bcls_ref, out_ref):
    sem = sem_ref[...] * (1.0 / _HW)
    C = sem.shape[1]

    h1 = jnp.maximum(snr_ref[0, 0] * w1_ref[...] + b1_ref[...], 0.0)
    sv = jnp.maximum(
        jnp.dot(h1, w2t_ref[...], preferred_element_type=jnp.float32)
        + b2_ref[...], 0.0)
    contrib = (
        jnp.dot(ch0_ref[...], wctc_ref[...], preferred_element_type=jnp.float32)
        + jnp.dot(sv, wcts_ref[...], preferred_element_type=jnp.float32))

    hid = jnp.maximum(
        jnp.dot(sem, wst_ref[...], preferred_element_type=jnp.float32)
        + contrib, 0.0)
    wts = jax.nn.sigmoid(
        jnp.dot(hid, wot_ref[...], preferred_element_type=jnp.float32))

    cr_c = jnp.clip(cr_ref[...], 0.001, 1.0)
    k = jnp.clip(jnp.round(cr_c * C), 1.0, float(C)).astype(jnp.int32)

    bits = jax.lax.bitcast_convert_type(wts, jnp.int32)

    def body(i, t):
        cand = t | (jnp.int32(1) << (jnp.int32(30) - i))
        cnt = jnp.sum((bits >= cand).astype(jnp.int32), axis=1, keepdims=True)
        return jnp.where(cnt >= k, cand, t)

    t = jax.lax.fori_loop(0, 31, body, jnp.zeros_like(k))
    mask = (bits >= t).astype(jnp.float32)

    pooled = sem * wts * mask
    out_ref[...] = (
        jnp.dot(pooled, wclst_ref[...], preferred_element_type=jnp.float32)
        + bcls_ref[...])


def kernel(x, snr_db, cr, channel_embed, snr_w1, snr_b1, snr_w2, snr_b2,
           Ws, Wc, Wo, Wcls, bcls):
    B, C, H, W = x.shape
    xt = jnp.transpose(x, (2, 3, 0, 1))

    sem_sum = pl.pallas_call(
        _sem_kernel,
        grid=(H // _HB,),
        in_specs=[pl.BlockSpec((_HB, W, B, C), lambda i: (i, 0, 0, 0))],
        out_specs=pl.BlockSpec((B, C), lambda i: (0, 0)),
        out_shape=jax.ShapeDtypeStruct((B, C), x.dtype),
    )(xt)

    crr = cr.reshape(B, 1)
    snr_sc = (jnp.asarray(snr_db, dtype=x.dtype) / 28.0).reshape(1, 1)
    w1r = snr_w1.T
    b1r = snr_b1.reshape(1, -1)
    w2t = snr_w2.T
    b2r = snr_b2.reshape(1, -1)
    ch0 = channel_embed[0].reshape(1, -1)
    e = channel_embed.shape[1]
    wct = Wc.T
    wctc, wcts = wct[:e], wct[e:]
    wst = Ws.T
    wot = Wo.T
    wclst = Wcls.T
    bclsr = bcls.reshape(1, -1)
    n_cls = Wcls.shape[0]
    hidden = Ws.shape[0]

    full = lambda shape: pl.BlockSpec(shape, lambda: (0,) * len(shape))
    return pl.pallas_call(
        _finish_kernel,
        in_specs=[
            full((B, C)),
            full((B, 1)),
            full((1, 1)),
            full((1, e)),
            full((1, e)),
            full((e, e)),
            full((1, e)),
            full((1, e)),
            full((e, hidden)),
            full((e, hidden)),
            full((C, hidden)),
            full((hidden, C)),
            full((C, n_cls)),
            full((1, n_cls)),
        ],
        out_specs=full((B, n_cls)),
        out_shape=jax.ShapeDtypeStruct((B, n_cls), x.dtype),
    )(sem_sum, crr, snr_sc, w1r, b1r, w2t, b2r, ch0, wctc, wcts, wst, wot,
      wclst, bclsr)


# speedup vs baseline: 14.2311x; 1.0359x over previous
import jax
import jax.numpy as jnp
from jax.experimental import pallas as pl
from jax.experimental.pallas import tpu as pltpu

_HW = 196
_HB = 2


def _fused_kernel(xt_ref, cr_ref, snr_ref, w1_ref, b1_ref, w2t_ref,
                  b2_ref, ch0_ref, wctc_ref, wcts_ref, wst_ref, wot_ref,
                  wcls_ref, bcls_ref, out_ref, sem_ref):
    i = pl.program_id(0)

    @pl.when(i == 0)
    def _():
        sem_ref[...] = jnp.zeros_like(sem_ref)

    acc = sem_ref[...]
    xb = xt_ref[...]
    for w in range(14):
        for h in range(_HB):
            acc = acc + xb[h, w]
    sem_ref[...] = acc

    @pl.when(i == pl.num_programs(0) - 1)
    def _():
        sem = acc * (1.0 / _HW)
        C = sem.shape[1]

        h1 = jnp.maximum(snr_ref[0, 0] * w1_ref[...] + b1_ref[...], 0.0)
        sv = jnp.maximum(
            jnp.dot(h1, w2t_ref[...], preferred_element_type=jnp.float32)
            + b2_ref[...], 0.0)
        contrib = (
            jnp.dot(ch0_ref[...], wctc_ref[...],
                    preferred_element_type=jnp.float32)
            + jnp.dot(sv, wcts_ref[...], preferred_element_type=jnp.float32))

        hid = jnp.maximum(
            jnp.dot(sem, wst_ref[...], preferred_element_type=jnp.float32)
            + contrib, 0.0)
        wts = jax.nn.sigmoid(
            jnp.dot(hid, wot_ref[...],
                    preferred_element_type=jnp.float32))

        cr_c = jnp.clip(cr_ref[...], 0.001, 1.0)
        k = jnp.clip(jnp.round(cr_c * C), 1.0, float(C)).astype(jnp.int32)

        bits = jax.lax.bitcast_convert_type(wts, jnp.int32)
        ones_col = jnp.ones((C, 1), dtype=jnp.float32)

        def body(j, t):
            cand = t | (jnp.int32(1) << (jnp.int32(29) - j))
            ge = (bits >= cand).astype(jnp.float32)
            cnt = jnp.dot(ge, ones_col,
                          preferred_element_type=jnp.float32)
            return jnp.where(cnt >= k.astype(jnp.float32), cand, t)

        t = jax.lax.fori_loop(0, 30, body, jnp.zeros_like(k))
        mask = (bits >= t).astype(jnp.float32)

        pooled = sem * wts * mask
        out_ref[...] = jax.lax.dot_general(
            pooled, wcls_ref[...],
            dimension_numbers=(((1,), (1,)), ((), ())),
            preferred_element_type=jnp.float32) + bcls_ref[...]


def kernel(x, snr_db, cr, channel_embed, snr_w1, snr_b1, snr_w2, snr_b2,
           Ws, Wc, Wo, Wcls, bcls):
    B, C, H, W = x.shape
    xt = jnp.transpose(x, (2, 3, 0, 1))

    crr = cr.reshape(B, 1)
    snr_sc = (jnp.asarray(snr_db, dtype=x.dtype) / 28.0).reshape(1, 1)
    w1r = snr_w1.T
    b1r = snr_b1.reshape(1, -1)
    w2t = snr_w2.T
    b2r = snr_b2.reshape(1, -1)
    ch0 = channel_embed[0].reshape(1, -1)
    e = channel_embed.shape[1]
    wct = Wc.T
    wctc, wcts = wct[:e], wct[e:]
    wst = Ws.T
    wot = Wo.T
    bclsr = bcls.reshape(1, -1)
    n_cls = Wcls.shape[0]
    hidden = Ws.shape[0]

    full = lambda shape: pl.BlockSpec(shape, lambda i: (0,) * len(shape))
    return pl.pallas_call(
        _fused_kernel,
        grid=(H // _HB,),
        in_specs=[
            pl.BlockSpec((_HB, W, B, C), lambda i: (i, 0, 0, 0)),
            full((B, 1)),
            full((1, 1)),
            full((1, e)),
            full((1, e)),
            full((e, e)),
            full((1, e)),
            full((1, e)),
            full((e, hidden)),
            full((e, hidden)),
            full((C, hidden)),
            full((hidden, C)),
            full((n_cls, C)),
            full((1, n_cls)),
        ],
        out_specs=full((B, n_cls)),
        out_shape=jax.ShapeDtypeStruct((B, n_cls), x.dtype),
        scratch_shapes=[pltpu.VMEM((B, C), jnp.float32)],
    )(xt, crr, snr_sc, w1r, b1r, w2t, b2r, ch0, wctc, wcts, wst, wot,
      Wcls, bclsr)


# speedup vs baseline: 15.6285x; 1.0982x over previous
import jax
import jax.numpy as jnp
from jax.experimental import pallas as pl
from jax.experimental.pallas import tpu as pltpu

_HW = 196
_HB = 2


def _fused_kernel(xt_ref, cr_ref, snr_ref, w1_ref, b1_ref, w2_ref,
                  b2_ref, ch0_ref, wc_ref, wst_ref, wot_ref,
                  wcls_ref, bcls_ref, out_ref, sem_ref):
    i = pl.program_id(0)

    @pl.when(i == 0)
    def _():
        sem_ref[...] = jnp.zeros_like(sem_ref)

    acc = sem_ref[...]
    xb = xt_ref[...]
    for w in range(14):
        for h in range(_HB):
            acc = acc + xb[h, w]
    sem_ref[...] = acc

    @pl.when(i == pl.num_programs(0) - 1)
    def _():
        sem = acc * (1.0 / _HW)
        C = sem.shape[1]

        rcontract = (((1,), (1,)), ((), ()))

        h1 = jnp.maximum(snr_ref[0, 0] * w1_ref[...] + b1_ref[...], 0.0)
        sv = jnp.maximum(
            jax.lax.dot_general(h1, w2_ref[...], rcontract,
                                preferred_element_type=jnp.float32)
            + b2_ref[...], 0.0)
        e = h1.shape[1]
        contrib = (
            jax.lax.dot_general(ch0_ref[...], wc_ref[:, :e], rcontract,
                                preferred_element_type=jnp.float32)
            + jax.lax.dot_general(sv, wc_ref[:, e:], rcontract,
                                  preferred_element_type=jnp.float32))

        hid = jnp.maximum(
            jnp.dot(sem, wst_ref[...], preferred_element_type=jnp.float32)
            + contrib, 0.0)
        wts = jax.nn.sigmoid(
            jnp.dot(hid, wot_ref[...],
                    preferred_element_type=jnp.float32))

        cr_c = jnp.clip(cr_ref[...], 0.001, 1.0)
        kf = jnp.clip(jnp.round(cr_c * C), 1.0, float(C))

        bits = jax.lax.bitcast_convert_type(wts, jnp.int32)
        ones_col = jnp.ones((C, 1), dtype=jnp.float32)

        def cnt(cand):
            ge = (bits >= cand).astype(jnp.float32)
            return jnp.dot(ge, ones_col,
                           preferred_element_type=jnp.float32)

        def body(j, t):
            bhi = jnp.int32(1) << (jnp.int32(29) - 2 * j)
            blo = bhi >> 1
            c1 = t | bhi
            c2 = t | blo
            c3 = c1 | blo
            take1 = cnt(c1) >= kf
            take2 = cnt(c2) >= kf
            take3 = cnt(c3) >= kf
            return jnp.where(take1, jnp.where(take3, c3, c1),
                             jnp.where(take2, c2, t))

        t = jax.lax.fori_loop(0, 15, body,
                              jnp.zeros(cr_c.shape, dtype=jnp.int32))
        mask = (bits >= t).astype(jnp.float32)

        pooled = sem * wts * mask
        out_ref[...] = jax.lax.dot_general(
            pooled, wcls_ref[...], rcontract,
            preferred_element_type=jnp.float32) + bcls_ref[...]


def kernel(x, snr_db, cr, channel_embed, snr_w1, snr_b1, snr_w2, snr_b2,
           Ws, Wc, Wo, Wcls, bcls):
    B, C, H, W = x.shape
    xt = jnp.transpose(x, (2, 3, 0, 1))

    crr = cr.reshape(B, 1)
    snr_sc = (jnp.asarray(snr_db, dtype=x.dtype) / 28.0).reshape(1, 1)
    w1r = snr_w1.T
    b1r = snr_b1.reshape(1, -1)
    b2r = snr_b2.reshape(1, -1)
    ch0 = channel_embed[0].reshape(1, -1)
    e = channel_embed.shape[1]
    bclsr = bcls.reshape(1, -1)
    n_cls = Wcls.shape[0]
    hidden = Ws.shape[0]

    full = lambda shape: pl.BlockSpec(shape, lambda i: (0,) * len(shape))
    return pl.pallas_call(
        _fused_kernel,
        grid=(H // _HB,),
        in_specs=[
            pl.BlockSpec((_HB, W, B, C), lambda i: (i, 0, 0, 0)),
            full((B, 1)),
            full((1, 1)),
            full((1, e)),
            full((1, e)),
            full((e, e)),
            full((1, e)),
            full((1, e)),
            full((hidden, 2 * e)),
            full((C, hidden)),
            full((hidden, C)),
            full((n_cls, C)),
            full((1, n_cls)),
        ],
        out_specs=full((B, n_cls)),
        out_shape=jax.ShapeDtypeStruct((B, n_cls), x.dtype),
        scratch_shapes=[pltpu.VMEM((B, C), jnp.float32)],
    )(xt, crr, snr_sc, w1r, b1r, snr_w2, b2r, ch0, Wc, Ws.T, Wo.T,
      Wcls, bclsr)
